# Initial kernel scaffold; baseline (speedup 1.0000x reference)
import jax
import jax.numpy as jnp
from jax.experimental import pallas as pl


def kernel(keypoint, pc_tgt, sn):
    raise NotImplementedError("write your pallas kernel here")


# speedup vs baseline: 1.7214x; 1.7214x over previous
import functools

import jax
import jax.numpy as jnp
from jax import lax
from jax.experimental import pallas as pl
from jax.experimental.pallas import tpu as pltpu
from jax.experimental.pallas import tpu_sc as plsc

B = 8
M = 512
N = 16384
NBLK = 8
BLK = N // NBLK
BIG_I32 = 2**30

NC = 2
NS = 16
NW = NC * NS
TOTAL_KP = B * M
KPW = TOTAL_KP // NW
L = 16


def _argmin_body(kp_ref, pct_ref, out_ref, runmin, runidx):
    b = pl.program_id(0)
    n = pl.program_id(1)

    @pl.when(n == 0)
    def _init():
        runmin[...] = jnp.full((1, M), jnp.inf, jnp.float32)
        runidx[...] = jnp.zeros((1, M), jnp.int32)

    acc = None
    for d in range(3):
        diff = pct_ref[0, :, d:d + 1] - kp_ref[0, d:d + 1, :]
        sq = diff * diff
        acc = sq if acc is None else acc + sq

    bmin = jnp.min(acc, axis=0, keepdims=True)
    iota0 = lax.broadcasted_iota(jnp.int32, (BLK, M), 0) + n * BLK
    cand = jnp.where(acc == bmin, iota0, BIG_I32)
    lidx = jnp.min(cand, axis=0, keepdims=True)

    better = bmin < runmin[...]
    runmin[...] = jnp.where(better, bmin, runmin[...])
    runidx[...] = jnp.where(better, lidx, runidx[...])

    @pl.when(n == NBLK - 1)
    def _fin():
        out_ref[...] = (runidx[...] + b * N)[:, None, :]


def _nn_indices(kp, pct):
    return pl.pallas_call(
        _argmin_body,
        grid=(B, NBLK),
        in_specs=[
            pl.BlockSpec((1, 3, M), lambda b, n: (b, 0, 0)),
            pl.BlockSpec((1, BLK, 3), lambda b, n: (b, n, 0)),
        ],
        out_specs=pl.BlockSpec((1, 1, M), lambda b, n: (b, 0, 0)),
        out_shape=jax.ShapeDtypeStruct((B, 1, M), jnp.int32),
        scratch_shapes=[
            pltpu.VMEM((1, M), jnp.float32),
            pltpu.VMEM((1, M), jnp.int32),
        ],
    )(kp, pct)


def _sc_loss_body(idx_hbm, px_hbm, py_hbm, pz_hbm, sx_hbm, sy_hbm, sz_hbm,
                  kx_hbm, ky_hbm, kz_hbm, out_hbm,
                  idx_v, gx_v, gy_v, gz_v, hx_v, hy_v, hz_v,
                  kx_v, ky_v, kz_v, accb, redb, totb, shared, sem):
    cid = lax.axis_index("c")
    sid = lax.axis_index("s")
    wid = cid * NS + sid
    base = wid * KPW

    pltpu.sync_copy(idx_hbm.at[pl.ds(base, KPW)], idx_v)
    pltpu.sync_copy(kx_hbm.at[pl.ds(base, KPW)], kx_v)
    pltpu.sync_copy(ky_hbm.at[pl.ds(base, KPW)], ky_v)
    pltpu.sync_copy(kz_hbm.at[pl.ds(base, KPW)], kz_v)
    hs = [pltpu.async_copy(src.at[idx_v], dst, sem)
          for src, dst in ((px_hbm, gx_v), (py_hbm, gy_v), (pz_hbm, gz_v),
                           (sx_hbm, hx_v), (sy_hbm, hy_v), (sz_hbm, hz_v))]
    for h in hs:
        h.wait()

    acc = jnp.zeros((L,), jnp.float32)
    for t in range(KPW // L):
        c = pl.ds(t * L, L)
        ux = kx_v[c] - gx_v[c]
        uy = ky_v[c] - gy_v[c]
        uz = kz_v[c] - gz_v[c]
        dot = ux * hx_v[c] + uy * hy_v[c] + uz * hz_v[c]
        u2 = ux * ux + uy * uy + uz * uz
        acc = acc + (dot * dot) / jnp.maximum(u2, jnp.float32(1e-30))

    accb[...] = acc
    pltpu.sync_copy(accb, shared.at[pl.ds(sid * L, L)])
    plsc.subcore_barrier()

    @pl.when(sid == 0)
    def _finish():
        pltpu.sync_copy(shared, redb)
        sv = redb[pl.ds(0, L)]
        for r in range(1, NS):
            sv = sv + redb[pl.ds(r * L, L)]
        total = sv[0]
        for l in range(1, L):
            total = total + sv[l]
        total = total * jnp.float32(1.0 / TOTAL_KP)
        totb[...] = jnp.full((L,), total, jnp.float32)
        pltpu.sync_copy(totb, out_hbm.at[pl.ds(cid * L, L)])


@functools.cache
def _make_sc_loss():
    return functools.partial(
        pl.kernel,
        mesh=plsc.VectorSubcoreMesh(core_axis_name="c", subcore_axis_name="s"),
        out_type=jax.ShapeDtypeStruct((2 * L,), jnp.float32),
        scratch_types=(
            [pltpu.VMEM((KPW,), jnp.int32)]
            + [pltpu.VMEM((KPW,), jnp.float32)] * 9
            + [
                pltpu.VMEM((L,), jnp.float32),
                pltpu.VMEM((NS * L,), jnp.float32),
                pltpu.VMEM((L,), jnp.float32),
                pltpu.VMEM_SHARED((NS * L,), jnp.float32),
                pltpu.SemaphoreType.DMA,
            ]
        ),
    )(_sc_loss_body)


def kernel(keypoint, pc_tgt, sn):
    kp = keypoint
    pct = jnp.transpose(pc_tgt, (0, 2, 1))
    snt = jnp.transpose(sn, (0, 2, 1))

    idx = _nn_indices(kp, pct).reshape(TOTAL_KP)

    px = pc_tgt[:, 0, :].reshape(B * N)
    py = pc_tgt[:, 1, :].reshape(B * N)
    pz = pc_tgt[:, 2, :].reshape(B * N)
    sx = sn[:, 0, :].reshape(B * N)
    sy = sn[:, 1, :].reshape(B * N)
    sz = sn[:, 2, :].reshape(B * N)
    kx = kp[:, 0, :].reshape(TOTAL_KP)
    ky = kp[:, 1, :].reshape(TOTAL_KP)
    kz = kp[:, 2, :].reshape(TOTAL_KP)

    out = _make_sc_loss()(idx, px, py, pz, sx, sy, sz, kx, ky, kz)
    return out[0] + out[L]


# speedup vs baseline: 2.1008x; 1.2204x over previous
import functools

import jax
import jax.numpy as jnp
from jax import lax
from jax.experimental import pallas as pl
from jax.experimental.pallas import tpu as pltpu
from jax.experimental.pallas import tpu_sc as plsc

B = 8
M = 512
N = 16384
NBLK = 8
BLK = N // NBLK
BIG_I32 = 2**30

NC = 2
NS = 16
NW = NC * NS
TOTAL_KP = B * M
KPW = TOTAL_KP // NW
WPB = M // KPW
L = 16


def _argmin_body(kpt_ref, pc_ref, out_ref, runmin, runidx):
    b = pl.program_id(0)
    n = pl.program_id(1)

    @pl.when(n == 0)
    def _init():
        runmin[...] = jnp.full((M, 1), jnp.inf, jnp.float32)
        runidx[...] = jnp.zeros((M, 1), jnp.int32)

    acc = None
    for d in range(3):
        diff = kpt_ref[0, :, d:d + 1] - pc_ref[0, d:d + 1, :]
        sq = diff * diff
        acc = sq if acc is None else acc + sq

    bmin = jnp.min(acc, axis=1, keepdims=True)
    iota1 = lax.broadcasted_iota(jnp.int32, (M, BLK), 1) + n * BLK
    cand = jnp.where(acc == bmin, iota1, BIG_I32)
    lidx = jnp.min(cand, axis=1, keepdims=True)

    better = bmin < runmin[...]
    runmin[...] = jnp.where(better, bmin, runmin[...])
    runidx[...] = jnp.where(better, lidx, runidx[...])

    @pl.when(n == NBLK - 1)
    def _fin():
        out_ref[...] = (runidx[...] + 3 * b * N)[None]


def _nn_indices(kpt, pc):
    return pl.pallas_call(
        _argmin_body,
        grid=(B, NBLK),
        in_specs=[
            pl.BlockSpec((1, M, 3), lambda b, n: (b, 0, 0)),
            pl.BlockSpec((1, 3, BLK), lambda b, n: (b, 0, n)),
        ],
        out_specs=pl.BlockSpec((1, M, 1), lambda b, n: (b, 0, 0)),
        out_shape=jax.ShapeDtypeStruct((B, M, 1), jnp.int32),
        scratch_shapes=[
            pltpu.VMEM((M, 1), jnp.float32),
            pltpu.VMEM((M, 1), jnp.int32),
        ],
    )(kpt, pc)


def _sc_loss_body(idx_hbm, pcf_hbm, snf_hbm, kpf_hbm, out_hbm,
                  idx_v, idy_v, idz_v, gx_v, gy_v, gz_v, hx_v, hy_v, hz_v,
                  kx_v, ky_v, kz_v, accb, redb, totb, shared, sem):
    cid = lax.axis_index("c")
    sid = lax.axis_index("s")
    wid = cid * NS + sid
    base = wid * KPW

    koff = 3 * (wid // WPB) * M + (wid % WPB) * KPW

    pltpu.sync_copy(idx_hbm.at[pl.ds(base, KPW)], idx_v)
    pltpu.sync_copy(kpf_hbm.at[pl.ds(koff, KPW)], kx_v)
    pltpu.sync_copy(kpf_hbm.at[pl.ds(koff + M, KPW)], ky_v)
    pltpu.sync_copy(kpf_hbm.at[pl.ds(koff + 2 * M, KPW)], kz_v)

    for t in range(KPW // L):
        c = pl.ds(t * L, L)
        v = idx_v[c]
        idy_v[c] = v + N
        idz_v[c] = v + 2 * N

    hs = [pltpu.async_copy(src.at[iv], dst, sem)
          for src, iv, dst in ((pcf_hbm, idx_v, gx_v),
                               (pcf_hbm, idy_v, gy_v),
                               (pcf_hbm, idz_v, gz_v),
                               (snf_hbm, idx_v, hx_v),
                               (snf_hbm, idy_v, hy_v),
                               (snf_hbm, idz_v, hz_v))]
    for h in hs:
        h.wait()

    acc = jnp.zeros((L,), jnp.float32)
    for t in range(KPW // L):
        c = pl.ds(t * L, L)
        ux = kx_v[c] - gx_v[c]
        uy = ky_v[c] - gy_v[c]
        uz = kz_v[c] - gz_v[c]
        dot = ux * hx_v[c] + uy * hy_v[c] + uz * hz_v[c]
        u2 = ux * ux + uy * uy + uz * uz
        acc = acc + (dot * dot) / jnp.maximum(u2, jnp.float32(1e-30))

    accb[...] = acc
    pltpu.sync_copy(accb, shared.at[pl.ds(sid * L, L)])
    plsc.subcore_barrier()

    @pl.when(sid == 0)
    def _finish():
        pltpu.sync_copy(shared, redb)
        sv = redb[pl.ds(0, L)]
        for r in range(1, NS):
            sv = sv + redb[pl.ds(r * L, L)]
        total = sv[0]
        for l in range(1, L):
            total = total + sv[l]
        total = total * jnp.float32(1.0 / TOTAL_KP)
        totb[...] = jnp.full((L,), total, jnp.float32)
        pltpu.sync_copy(totb, out_hbm.at[pl.ds(cid * L, L)])


@functools.cache
def _make_sc_loss():
    return functools.partial(
        pl.kernel,
        mesh=plsc.VectorSubcoreMesh(core_axis_name="c", subcore_axis_name="s"),
        out_type=jax.ShapeDtypeStruct((2 * L,), jnp.float32),
        scratch_types=(
            [pltpu.VMEM((KPW,), jnp.int32)] * 3
            + [pltpu.VMEM((KPW,), jnp.float32)] * 9
            + [
                pltpu.VMEM((L,), jnp.float32),
                pltpu.VMEM((NS * L,), jnp.float32),
                pltpu.VMEM((L,), jnp.float32),
                pltpu.VMEM_SHARED((NS * L,), jnp.float32),
                pltpu.SemaphoreType.DMA,
            ]
        ),
    )(_sc_loss_body)


def kernel(keypoint, pc_tgt, sn):
    kpt = jnp.transpose(keypoint, (0, 2, 1))
    idx = _nn_indices(kpt, pc_tgt).reshape(TOTAL_KP)
    out = _make_sc_loss()(
        idx,
        pc_tgt.reshape(3 * B * N),
        sn.reshape(3 * B * N),
        keypoint.reshape(3 * B * M),
    )
    return out[0] + out[L]


# speedup vs baseline: 2.5051x; 1.1924x over previous
import functools

import jax
import jax.numpy as jnp
from jax import lax
from jax.experimental import pallas as pl
from jax.experimental.pallas import tpu as pltpu
from jax.experimental.pallas import tpu_sc as plsc

B = 8
M = 512
N = 16384
NBLK = 8
BLK = N // NBLK
BIG_I32 = 2**30

NC = 2
NS = 16
NW = NC * NS
TOTAL_KP = B * M
KPW = TOTAL_KP // NW
WPB = M // KPW
L = 16


IDX_BITS = 11
IDX_MASK = (1 << IDX_BITS) - 1


def _argmin_body(kpt_ref, pc_ref, out_ref, runkey, runblk):
    b = pl.program_id(0)
    n = pl.program_id(1)

    @pl.when(n == 0)
    def _init():
        runkey[...] = jnp.full((M, 1), jnp.inf, jnp.float32)
        runblk[...] = jnp.zeros((M, 1), jnp.int32)

    acc = None
    for d in range(3):
        diff = kpt_ref[0, :, d:d + 1] - pc_ref[0, d:d + 1, :]
        sq = diff * diff
        acc = sq if acc is None else acc + sq

    iota1 = lax.broadcasted_iota(jnp.int32, (M, BLK), 1)
    bits = lax.bitcast_convert_type(acc, jnp.int32)
    key = lax.bitcast_convert_type((bits & ~IDX_MASK) | iota1, jnp.float32)
    bkey = jnp.min(key, axis=1, keepdims=True)

    better = bkey < runkey[...]
    runkey[...] = jnp.where(better, bkey, runkey[...])
    runblk[...] = jnp.where(better, jnp.full((M, 1), n, jnp.int32),
                            runblk[...])

    @pl.when(n == NBLK - 1)
    def _fin():
        lidx = lax.bitcast_convert_type(runkey[...], jnp.int32) & IDX_MASK
        out_ref[...] = (runblk[...] * BLK + lidx + 3 * b * N)[None]


def _nn_indices(kpt, pc):
    return pl.pallas_call(
        _argmin_body,
        grid=(B, NBLK),
        in_specs=[
            pl.BlockSpec((1, M, 3), lambda b, n: (b, 0, 0)),
            pl.BlockSpec((1, 3, BLK), lambda b, n: (b, 0, n)),
        ],
        out_specs=pl.BlockSpec((1, M, 1), lambda b, n: (b, 0, 0)),
        out_shape=jax.ShapeDtypeStruct((B, M, 1), jnp.int32),
        scratch_shapes=[
            pltpu.VMEM((M, 1), jnp.float32),
            pltpu.VMEM((M, 1), jnp.int32),
        ],
    )(kpt, pc)


def _sc_loss_body(idx_hbm, pcf_hbm, snf_hbm, kpf_hbm, out_hbm,
                  idx_v, idy_v, idz_v, gx_v, gy_v, gz_v, hx_v, hy_v, hz_v,
                  kx_v, ky_v, kz_v, accb, redb, totb, shared, sem):
    cid = lax.axis_index("c")
    sid = lax.axis_index("s")
    wid = cid * NS + sid
    base = wid * KPW

    koff = 3 * (wid // WPB) * M + (wid % WPB) * KPW

    pltpu.sync_copy(idx_hbm.at[pl.ds(base, KPW)], idx_v)
    pltpu.sync_copy(kpf_hbm.at[pl.ds(koff, KPW)], kx_v)
    pltpu.sync_copy(kpf_hbm.at[pl.ds(koff + M, KPW)], ky_v)
    pltpu.sync_copy(kpf_hbm.at[pl.ds(koff + 2 * M, KPW)], kz_v)

    for t in range(KPW // L):
        c = pl.ds(t * L, L)
        v = idx_v[c]
        idy_v[c] = v + N
        idz_v[c] = v + 2 * N

    hs = [pltpu.async_copy(src.at[iv], dst, sem)
          for src, iv, dst in ((pcf_hbm, idx_v, gx_v),
                               (pcf_hbm, idy_v, gy_v),
                               (pcf_hbm, idz_v, gz_v),
                               (snf_hbm, idx_v, hx_v),
                               (snf_hbm, idy_v, hy_v),
                               (snf_hbm, idz_v, hz_v))]
    for h in hs:
        h.wait()

    acc = jnp.zeros((L,), jnp.float32)
    for t in range(KPW // L):
        c = pl.ds(t * L, L)
        ux = kx_v[c] - gx_v[c]
        uy = ky_v[c] - gy_v[c]
        uz = kz_v[c] - gz_v[c]
        dot = ux * hx_v[c] + uy * hy_v[c] + uz * hz_v[c]
        u2 = ux * ux + uy * uy + uz * uz
        acc = acc + (dot * dot) / jnp.maximum(u2, jnp.float32(1e-30))

    accb[...] = acc
    pltpu.sync_copy(accb, shared.at[pl.ds(sid * L, L)])
    plsc.subcore_barrier()

    @pl.when(sid == 0)
    def _finish():
        pltpu.sync_copy(shared, redb)
        sv = redb[pl.ds(0, L)]
        for r in range(1, NS):
            sv = sv + redb[pl.ds(r * L, L)]
        total = sv[0]
        for l in range(1, L):
            total = total + sv[l]
        total = total * jnp.float32(1.0 / TOTAL_KP)
        totb[...] = jnp.full((L,), total, jnp.float32)
        pltpu.sync_copy(totb, out_hbm.at[pl.ds(cid * L, L)])


@functools.cache
def _make_sc_loss():
    return functools.partial(
        pl.kernel,
        mesh=plsc.VectorSubcoreMesh(core_axis_name="c", subcore_axis_name="s"),
        out_type=jax.ShapeDtypeStruct((2 * L,), jnp.float32),
        scratch_types=(
            [pltpu.VMEM((KPW,), jnp.int32)] * 3
            + [pltpu.VMEM((KPW,), jnp.float32)] * 9
            + [
                pltpu.VMEM((L,), jnp.float32),
                pltpu.VMEM((NS * L,), jnp.float32),
                pltpu.VMEM((L,), jnp.float32),
                pltpu.VMEM_SHARED((NS * L,), jnp.float32),
                pltpu.SemaphoreType.DMA,
            ]
        ),
    )(_sc_loss_body)


def kernel(keypoint, pc_tgt, sn):
    kpt = jnp.transpose(keypoint, (0, 2, 1))
    idx = _nn_indices(kpt, pc_tgt).reshape(TOTAL_KP)
    out = _make_sc_loss()(
        idx,
        pc_tgt.reshape(3 * B * N),
        sn.reshape(3 * B * N),
        keypoint.reshape(3 * B * M),
    )
    return out[0] + out[L]


# speedup vs baseline: 2.7805x; 1.1100x over previous
import functools

import jax
import jax.numpy as jnp
from jax import lax
from jax.experimental import pallas as pl
from jax.experimental.pallas import tpu as pltpu
from jax.experimental.pallas import tpu_sc as plsc

B = 8
M = 512
N = 16384
NBLK = 8
BLK = N // NBLK
BIG_I32 = 2**30

NC = 2
NS = 16
NW = NC * NS
TOTAL_KP = B * M
KPW = TOTAL_KP // NW
WPB = M // KPW
L = 16


IDX_BITS = 11
IDX_MASK = (1 << IDX_BITS) - 1


def _argmin_body(kpt_ref, pc_ref, out_ref, runkey, runblk):
    b = pl.program_id(0)
    n = pl.program_id(1)

    @pl.when(n == 0)
    def _init():
        runkey[...] = jnp.full((M, 1), jnp.inf, jnp.float32)
        runblk[...] = jnp.zeros((M, 1), jnp.int32)

    pcx = pc_ref[0, 0:1, :]
    pcy = pc_ref[0, 1:2, :]
    pcz = pc_ref[0, 2:3, :]
    pcn2 = pcx * pcx + pcy * pcy + pcz * pcz
    acc = pcn2 + kpt_ref[0, :, 0:1] * pcx
    acc = acc + kpt_ref[0, :, 1:2] * pcy
    acc = acc + kpt_ref[0, :, 2:3] * pcz

    iota1 = lax.broadcasted_iota(jnp.int32, (M, BLK), 1)
    bits = lax.bitcast_convert_type(acc, jnp.int32)
    key = lax.bitcast_convert_type((bits & ~IDX_MASK) | iota1, jnp.float32)
    bkey = jnp.min(key, axis=1, keepdims=True)

    better = bkey < runkey[...]
    runkey[...] = jnp.where(better, bkey, runkey[...])
    runblk[...] = jnp.where(better, jnp.full((M, 1), n, jnp.int32),
                            runblk[...])

    @pl.when(n == NBLK - 1)
    def _fin():
        lidx = lax.bitcast_convert_type(runkey[...], jnp.int32) & IDX_MASK
        out_ref[...] = (runblk[...] * BLK + lidx + 3 * b * N)[None]


def _nn_indices(kpt, pc):
    return pl.pallas_call(
        _argmin_body,
        grid=(B, NBLK),
        in_specs=[
            pl.BlockSpec((1, M, 3), lambda b, n: (b, 0, 0)),
            pl.BlockSpec((1, 3, BLK), lambda b, n: (b, 0, n)),
        ],
        out_specs=pl.BlockSpec((1, M, 1), lambda b, n: (b, 0, 0)),
        out_shape=jax.ShapeDtypeStruct((B, M, 1), jnp.int32),
        scratch_shapes=[
            pltpu.VMEM((M, 1), jnp.float32),
            pltpu.VMEM((M, 1), jnp.int32),
        ],
    )(kpt, pc)


def _sc_loss_body(idx_hbm, pcf_hbm, snf_hbm, kpf_hbm, out_hbm,
                  idx_v, idy_v, idz_v, gx_v, gy_v, gz_v, hx_v, hy_v, hz_v,
                  kx_v, ky_v, kz_v, accb, redb, totb, shared, sem):
    cid = lax.axis_index("c")
    sid = lax.axis_index("s")
    wid = cid * NS + sid
    base = wid * KPW

    koff = 3 * (wid // WPB) * M + (wid % WPB) * KPW

    pltpu.sync_copy(idx_hbm.at[pl.ds(base, KPW)], idx_v)
    pltpu.sync_copy(kpf_hbm.at[pl.ds(koff, KPW)], kx_v)
    pltpu.sync_copy(kpf_hbm.at[pl.ds(koff + M, KPW)], ky_v)
    pltpu.sync_copy(kpf_hbm.at[pl.ds(koff + 2 * M, KPW)], kz_v)

    for t in range(KPW // L):
        c = pl.ds(t * L, L)
        v = idx_v[c]
        idy_v[c] = v + N
        idz_v[c] = v + 2 * N

    hs = [pltpu.async_copy(src.at[iv], dst, sem)
          for src, iv, dst in ((pcf_hbm, idx_v, gx_v),
                               (pcf_hbm, idy_v, gy_v),
                               (pcf_hbm, idz_v, gz_v),
                               (snf_hbm, idx_v, hx_v),
                               (snf_hbm, idy_v, hy_v),
                               (snf_hbm, idz_v, hz_v))]
    for h in hs:
        h.wait()

    acc = jnp.zeros((L,), jnp.float32)
    for t in range(KPW // L):
        c = pl.ds(t * L, L)
        ux = kx_v[c] - gx_v[c]
        uy = ky_v[c] - gy_v[c]
        uz = kz_v[c] - gz_v[c]
        dot = ux * hx_v[c] + uy * hy_v[c] + uz * hz_v[c]
        u2 = ux * ux + uy * uy + uz * uz
        acc = acc + (dot * dot) / jnp.maximum(u2, jnp.float32(1e-30))

    accb[...] = acc
    pltpu.sync_copy(accb, shared.at[pl.ds(sid * L, L)])
    plsc.subcore_barrier()

    @pl.when(sid == 0)
    def _finish():
        pltpu.sync_copy(shared, redb)
        sv = redb[pl.ds(0, L)]
        for r in range(1, NS):
            sv = sv + redb[pl.ds(r * L, L)]
        total = sv[0]
        for l in range(1, L):
            total = total + sv[l]
        total = total * jnp.float32(1.0 / TOTAL_KP)
        totb[...] = jnp.full((L,), total, jnp.float32)
        pltpu.sync_copy(totb, out_hbm.at[pl.ds(cid * L, L)])


@functools.cache
def _make_sc_loss():
    return functools.partial(
        pl.kernel,
        mesh=plsc.VectorSubcoreMesh(core_axis_name="c", subcore_axis_name="s"),
        out_type=jax.ShapeDtypeStruct((2 * L,), jnp.float32),
        scratch_types=(
            [pltpu.VMEM((KPW,), jnp.int32)] * 3
            + [pltpu.VMEM((KPW,), jnp.float32)] * 9
            + [
                pltpu.VMEM((L,), jnp.float32),
                pltpu.VMEM((NS * L,), jnp.float32),
                pltpu.VMEM((L,), jnp.float32),
                pltpu.VMEM_SHARED((NS * L,), jnp.float32),
                pltpu.SemaphoreType.DMA,
            ]
        ),
    )(_sc_loss_body)


def kernel(keypoint, pc_tgt, sn):
    kptm2 = jnp.transpose(keypoint * -2.0, (0, 2, 1))
    idx = _nn_indices(kptm2, pc_tgt).reshape(TOTAL_KP)
    out = _make_sc_loss()(
        idx,
        pc_tgt.reshape(3 * B * N),
        sn.reshape(3 * B * N),
        keypoint.reshape(3 * B * M),
    )
    return out[0] + out[L]
